# Initial kernel scaffold; baseline (speedup 1.0000x reference)
import jax
import jax.numpy as jnp
from jax.experimental import pallas as pl


def kernel(input_ids, token_table, pos_table):
    raise NotImplementedError("write your pallas kernel here")


# speedup vs baseline: 2.2823x; 2.2823x over previous
import jax
import jax.numpy as jnp
from jax import lax
from jax.experimental import pallas as pl
from jax.experimental.pallas import tpu as pltpu
from jax.experimental.pallas import tpu_sc as plsc

VOCAB = 100000
EMBED = 64
NUM_POS = 200
BATCH = 4096
SEQ = 200

NC = 2
NS = 16
NW = NC * NS

TOKENS = BATCH * SEQ
TOK_PER_W = TOKENS // NW
CHUNK = 128
CHUNKS_PER_W = TOK_PER_W // CHUNK


def _body(ids_hbm, pos2_hbm, table_hbm, out_hbm, ids_v, pos_v, rows_v, sem):
    wid = lax.axis_index("s") * NC + lax.axis_index("c")
    chunk0 = wid * CHUNKS_PER_W

    pltpu.sync_copy(ids_hbm.at[pl.ds(chunk0, CHUNKS_PER_W)], ids_v)
    pltpu.sync_copy(pos2_hbm, pos_v)

    def chunk_step(c, _):
        pltpu.async_copy(table_hbm.at[ids_v.at[c]], rows_v, sem).wait()
        s0 = ((chunk0 + c) * CHUNK) % NUM_POS

        def row_step(r, _):
            for d in range(EMBED // 16):
                sl = pl.ds(d * 16, 16)
                rows_v[r, sl] = rows_v[r, sl] + pos_v[s0 + r, sl]
            return ()

        lax.fori_loop(0, CHUNK, row_step, (), unroll=4)
        pltpu.sync_copy(rows_v, out_hbm.at[pl.ds((chunk0 + c) * CHUNK, CHUNK)])
        return ()

    lax.fori_loop(0, CHUNKS_PER_W, chunk_step, ())


@jax.jit
def _run(ids_2d, token_table, pos2):
    kern = pl.kernel(
        _body,
        out_type=jax.ShapeDtypeStruct((TOKENS, EMBED), jnp.float32),
        mesh=plsc.VectorSubcoreMesh(core_axis_name="c", subcore_axis_name="s"),
        scratch_types=[
            pltpu.VMEM((CHUNKS_PER_W, CHUNK), jnp.int32),
            pltpu.VMEM((2 * NUM_POS, EMBED), jnp.float32),
            pltpu.VMEM((CHUNK, EMBED), jnp.float32),
            pltpu.SemaphoreType.DMA,
        ],
        compiler_params=pltpu.CompilerParams(use_tc_tiling_on_sc=False),
    )
    return kern(ids_2d, pos2, token_table)


def kernel(input_ids, token_table, pos_table):
    ids_2d = input_ids.astype(jnp.int32).reshape(TOKENS // CHUNK, CHUNK)
    pos2 = jnp.concatenate([pos_table, pos_table], axis=0)
    out = _run(ids_2d, token_table, pos2)
    return out.reshape(BATCH, SEQ, EMBED)


# speedup vs baseline: 2.6011x; 1.1397x over previous
import jax
import jax.numpy as jnp
from jax import lax
from jax.experimental import pallas as pl
from jax.experimental.pallas import tpu as pltpu
from jax.experimental.pallas import tpu_sc as plsc

VOCAB = 100000
EMBED = 64
NUM_POS = 200
BATCH = 4096
SEQ = 200

NC = 2
NS = 16
NW = NC * NS

TOKENS = BATCH * SEQ
TOK_PER_W = TOKENS // NW
CHUNK = 128
CHUNKS_PER_W = TOK_PER_W // CHUNK


def _body(ids_hbm, pos2_hbm, table_hbm, out_hbm,
          ids_v, pos_v, rows0, rows1, gs0, gs1, ss0, ss1):
    wid = lax.axis_index("s") * NC + lax.axis_index("c")
    chunk0 = wid * CHUNKS_PER_W

    pltpu.sync_copy(ids_hbm.at[pl.ds(chunk0, CHUNKS_PER_W)], ids_v)
    pltpu.sync_copy(pos2_hbm, pos_v)

    def gather(c, rows, sem):
        return pltpu.make_async_copy(table_hbm.at[ids_v.at[c]], rows, sem)

    def store(c, rows, sem):
        dst = out_hbm.at[pl.ds((chunk0 + c) * CHUNK, CHUNK)]
        return pltpu.make_async_copy(rows, dst, sem)

    def add_pos(c, rows):
        s0 = ((chunk0 + c) * CHUNK) % NUM_POS

        def row_step(r, _):
            for d in range(EMBED // 16):
                sl = pl.ds(d * 16, 16)
                rows[r, sl] = rows[r, sl] + pos_v[s0 + r, sl]
            return ()

        lax.fori_loop(0, CHUNK, row_step, (), unroll=8)

    gather(0, rows0, gs0).start()
    gather(0, rows0, gs0).wait()
    gather(1, rows1, gs1).start()
    add_pos(0, rows0)
    store(0, rows0, ss0).start()

    def pair(t, _):
        c = 2 * t + 1
        gather(c, rows1, gs1).wait()
        store(c - 1, rows0, ss0).wait()
        gather(c + 1, rows0, gs0).start()
        add_pos(c, rows1)
        store(c, rows1, ss1).start()
        gather(c + 1, rows0, gs0).wait()
        store(c, rows1, ss1).wait()
        gather(c + 2, rows1, gs1).start()
        add_pos(c + 1, rows0)
        store(c + 1, rows0, ss0).start()
        return ()

    lax.fori_loop(0, (CHUNKS_PER_W - 2) // 2, pair, ())

    c_last = CHUNKS_PER_W - 1
    gather(c_last, rows1, gs1).wait()
    add_pos(c_last, rows1)
    store(c_last - 1, rows0, ss0).wait()
    store(c_last, rows1, ss1).start()
    store(c_last, rows1, ss1).wait()


@jax.jit
def _run(ids_2d, token_table, pos2):
    kern = pl.kernel(
        _body,
        out_type=jax.ShapeDtypeStruct((TOKENS, EMBED), jnp.float32),
        mesh=plsc.VectorSubcoreMesh(core_axis_name="c", subcore_axis_name="s"),
        scratch_types=[
            pltpu.VMEM((CHUNKS_PER_W, CHUNK), jnp.int32),
            pltpu.VMEM((2 * NUM_POS, EMBED), jnp.float32),
            pltpu.VMEM((CHUNK, EMBED), jnp.float32),
            pltpu.VMEM((CHUNK, EMBED), jnp.float32),
            pltpu.SemaphoreType.DMA,
            pltpu.SemaphoreType.DMA,
            pltpu.SemaphoreType.DMA,
            pltpu.SemaphoreType.DMA,
        ],
        compiler_params=pltpu.CompilerParams(use_tc_tiling_on_sc=False),
    )
    return kern(ids_2d, pos2, token_table)


def kernel(input_ids, token_table, pos_table):
    ids_2d = input_ids.astype(jnp.int32).reshape(TOKENS // CHUNK, CHUNK)
    pos2 = jnp.concatenate([pos_table, pos_table], axis=0)
    out = _run(ids_2d, token_table, pos2)
    return out.reshape(BATCH, SEQ, EMBED)
